# baseline (speedup 1.0000x reference)
import functools

import jax
import jax.numpy as jnp
from jax import lax
from jax.experimental import pallas as pl
from jax.experimental.pallas import tpu as pltpu
from jax.experimental.pallas import tpu_sc as plsc

N_NODES = 10000
N_EDGES = 320000
D = 128

NC = 2
NS = 16
CHUNK = 128

NT = NC * NS
IBLK = 8
PAD_UNIT = NT * CHUNK * IBLK * 2
EDGES_PAD = ((N_EDGES + PAD_UNIT - 1) // PAD_UNIT) * PAD_UNIT
EDGES_PER_TILE = EDGES_PAD // NT
CPT = EDGES_PER_TILE // CHUNK
BLOCKS = CPT // IBLK

ACC_ROWS = 10240
ZERO_PER_TILE = ACC_ROWS // NS
WRITE_PER_TILE = (N_NODES // NS) // 8 * 8
WRITE_TAIL = N_NODES - NS * WRITE_PER_TILE


def _linear_body(feat_ref, w_ref, b_ref, out_ref):
    h = lax.dot_general(
        feat_ref[...], w_ref[...],
        dimension_numbers=(((1,), (1,)), ((), ())),
        preferred_element_type=jnp.float32,
    )
    out_ref[...] = h + b_ref[...]


def _linear(feat, W, b):
    rb = 1000
    return pl.pallas_call(
        _linear_body,
        grid=(N_NODES // rb,),
        in_specs=[
            pl.BlockSpec((rb, D), lambda i: (i, 0)),
            pl.BlockSpec((D, D), lambda i: (0, 0)),
            pl.BlockSpec((1, D), lambda i: (0, 0)),
        ],
        out_specs=pl.BlockSpec((rb, D), lambda i: (i, 0)),
        out_shape=jax.ShapeDtypeStruct((N_NODES, D), jnp.float32),
    )(feat, W, b.reshape(1, D))


def _mp_body(h, zeros, src3, dst3, out, srcA, dstA, srcB, dstB,
             rows, acc, semG, semS, semSA, semDA, semSB, semDB):
    c = lax.axis_index("c")
    s = lax.axis_index("s")
    tid = c * NS + s

    z0 = s * ZERO_PER_TILE
    pltpu.sync_copy(zeros.at[pl.ds(z0, ZERO_PER_TILE)],
                    acc.at[pl.ds(z0, ZERO_PER_TILE)])

    def load_idx(blk, sbuf, dbuf, ssem, dsem):
        pltpu.async_copy(src3.at[tid, pl.ds(blk * IBLK, IBLK)], sbuf, ssem)
        pltpu.async_copy(dst3.at[tid, pl.ds(blk * IBLK, IBLK)], dbuf, dsem)

    def wait_idx(sbuf, dbuf, ssem, dsem):
        pltpu.make_async_copy(src3.at[tid, pl.ds(0, IBLK)], sbuf, ssem).wait()
        pltpu.make_async_copy(dst3.at[tid, pl.ds(0, IBLK)], dbuf, dsem).wait()

    def fire_g(idx_row, p):
        pltpu.async_copy(h.at[idx_row], rows[p], semG[p])

    def wait_g(p):
        pltpu.make_async_copy(h.at[srcA.at[0]], rows[p], semG[p]).wait()

    def fire_s(idx_row, p):
        pltpu.async_copy(rows[p], acc.at[idx_row], semS[p], add=True)

    def wait_s(p):
        pltpu.make_async_copy(rows[p], acc.at[dstA.at[0]], semS[p]).wait()

    load_idx(0, srcA, dstA, semSA, semDA)
    plsc.subcore_barrier()

    def half(blk, sbuf, dbuf, osbuf, odbuf, ssem, dsem, ossem, odsem,
             first=False, last=False):
        wait_idx(sbuf, dbuf, ssem, dsem)
        for j in range(IBLK):
            p = j % 2
            q = (j + 1) % 2
            if not (first and j < 2):
                wait_s(p)
            fire_g(sbuf.at[j], p)
            if not (first and j == 0):
                wait_g(q)
                if j > 0:
                    fire_s(dbuf.at[j - 1], q)
                else:
                    fire_s(odbuf.at[IBLK - 1], q)
            if j == 1 and not last:
                load_idx(blk + 1, osbuf, odbuf, ossem, odsem)

    half(0, srcA, dstA, srcB, dstB, semSA, semDA, semSB, semDB, first=True)

    @pl.loop(0, (BLOCKS - 2) // 2)
    def _(bp):
        blk = 1 + 2 * bp
        half(blk, srcB, dstB, srcA, dstA, semSB, semDB, semSA, semDA)
        half(blk + 1, srcA, dstA, srcB, dstB, semSA, semDA, semSB, semDB)

    half(BLOCKS - 1, srcB, dstB, srcA, dstA, semSB, semDB, semSA, semDA,
         last=True)

    wait_g(1)
    fire_s(dstB.at[IBLK - 1], 1)
    wait_s(0)
    wait_s(1)

    plsc.subcore_barrier()
    w0 = s * WRITE_PER_TILE
    pltpu.sync_copy(acc.at[pl.ds(w0, WRITE_PER_TILE)],
                    out.at[c, pl.ds(w0, WRITE_PER_TILE)])

    @pl.when(s == 0)
    def _():
        t0 = NS * WRITE_PER_TILE
        pltpu.sync_copy(acc.at[pl.ds(t0, WRITE_TAIL)],
                        out.at[c, pl.ds(t0, WRITE_TAIL)])


@functools.partial(
    pl.kernel,
    out_type=jax.ShapeDtypeStruct((NC, N_NODES, D), jnp.float32),
    mesh=plsc.VectorSubcoreMesh(core_axis_name="c", subcore_axis_name="s"),
    scratch_types=[
        pltpu.VMEM((IBLK, CHUNK), jnp.int32),
        pltpu.VMEM((IBLK, CHUNK), jnp.int32),
        pltpu.VMEM((IBLK, CHUNK), jnp.int32),
        pltpu.VMEM((IBLK, CHUNK), jnp.int32),
        [pltpu.VMEM((CHUNK, D), jnp.float32)] * 2,
        pltpu.VMEM_SHARED((ACC_ROWS, D), jnp.float32),
        [pltpu.SemaphoreType.DMA] * 2,
        [pltpu.SemaphoreType.DMA] * 2,
        pltpu.SemaphoreType.DMA,
        pltpu.SemaphoreType.DMA,
        pltpu.SemaphoreType.DMA,
        pltpu.SemaphoreType.DMA,
    ],
)
def _message_passing(h, zeros, src3, dst3, out, srcA, dstA, srcB, dstB,
                     rows, acc, semG, semS, semSA, semDA, semSB, semDB):
    _mp_body(h, zeros, src3, dst3, out, srcA, dstA, srcB, dstB,
             rows, acc, semG, semS, semSA, semDA, semSB, semDB)


def _combine_body(p_ref, out_ref):
    out_ref[...] = p_ref[0] + p_ref[1]


def _combine(p):
    rb = 1000
    return pl.pallas_call(
        _combine_body,
        grid=(N_NODES // rb,),
        in_specs=[pl.BlockSpec((NC, rb, D), lambda i: (0, i, 0))],
        out_specs=pl.BlockSpec((rb, D), lambda i: (i, 0)),
        out_shape=jax.ShapeDtypeStruct((N_NODES, D), jnp.float32),
    )(p)


@jax.jit
def kernel(feat, edge_index, W, b):
    h = _linear(feat, W, b)
    npad = EDGES_PAD - N_EDGES
    src3 = jnp.concatenate(
        [edge_index[0], jnp.zeros((npad,), jnp.int32)]).reshape(NT, CPT, CHUNK)
    dst3 = jnp.concatenate(
        [edge_index[1], jnp.full((npad,), N_NODES, jnp.int32)]
    ).reshape(NT, CPT, CHUNK)
    zeros = jnp.zeros((ACC_ROWS, D), jnp.float32)
    return _combine(_message_passing(h, zeros, src3, dst3))
